# baseline (speedup 1.0000x reference)
import jax
import jax.numpy as jnp
from jax import lax
from jax.experimental import pallas as pl
from jax.experimental.pallas import tpu as pltpu


def _linear_block(h_ref, wt_ref, b_ref, o_ref):
    acc = jnp.dot(h_ref[:, :], wt_ref[:, :], preferred_element_type=jnp.float32)
    o_ref[:, :] = acc + b_ref[:, :]


def kernel(h, edge_index, edge_sh, edge_radial, n_atoms, W, b):
    n, d = h.shape
    block = 400 if n % 400 == 0 else 8
    grid = pl.cdiv(n, block)
    wt = W.T
    b2 = b.reshape(1, d)
    return pl.pallas_call(
        _linear_block,
        grid=(grid,),
        in_specs=[
            pl.BlockSpec((block, d), lambda i: (i, 0)),
            pl.BlockSpec((d, d), lambda i: (0, 0)),
            pl.BlockSpec((1, d), lambda i: (0, 0)),
        ],
        out_specs=pl.BlockSpec((block, d), lambda i: (i, 0)),
        out_shape=jax.ShapeDtypeStruct((n, d), jnp.float32),
        compiler_params=pltpu.CompilerParams(
            dimension_semantics=("parallel",),
        ),
    )(h, wt, b2)


# speedup vs baseline: 2.5868x; 2.5868x over previous
import jax
import jax.numpy as jnp
from jax import lax
from jax.experimental import pallas as pl
from jax.experimental.pallas import tpu as pltpu


def _linear_block(h_ref, wt_ref, b_ref, o_ref):
    acc = jnp.dot(h_ref[:, :], wt_ref[:, :], preferred_element_type=jnp.float32)
    o_ref[:, :] = acc + b_ref[:, :]


def kernel(h, edge_index, edge_sh, edge_radial, n_atoms, W, b):
    n, d = h.shape
    block = n
    grid = pl.cdiv(n, block)
    wt = W.T
    b2 = b.reshape(1, d)
    return pl.pallas_call(
        _linear_block,
        grid=(grid,),
        in_specs=[
            pl.BlockSpec((block, d), lambda i: (i, 0)),
            pl.BlockSpec((d, d), lambda i: (0, 0)),
            pl.BlockSpec((1, d), lambda i: (0, 0)),
        ],
        out_specs=pl.BlockSpec((block, d), lambda i: (i, 0)),
        out_shape=jax.ShapeDtypeStruct((n, d), jnp.float32),
        compiler_params=pltpu.CompilerParams(
            dimension_semantics=("parallel",),
        ),
    )(h, wt, b2)
